# baseline (speedup 1.0000x reference)
import jax
import jax.numpy as jnp
from jax import lax
from jax.experimental import pallas as pl
from jax.experimental.pallas import tpu as pltpu
from jax.experimental.pallas import tpu_sc as plsc

_INFO = plsc.get_sparse_core_info()
_NC = _INFO.num_cores
_NS = _INFO.num_subcores
_NT = _NC * _NS
_CHUNK = 128
_NBUF = 8
_H = 16
_DW = 8


def _deg_body(srcm, dstm, srct, dstt, ones_hbm, zeros_hbm, out_hbm,
              ones_v, src_v, dst_v, st_v, dt_v, acc_o, acc_i,
              sem, sem2, sem3, sem4):
    c = lax.axis_index("c")
    s = lax.axis_index("s")
    wid = c * _NS + s
    np_ = acc_o.shape[0]
    rpt = np_ // _NS
    full = src_v.shape[0]
    pltpu.sync_copy(zeros_hbm.at[pl.ds(s * rpt, rpt)],
                    acc_o.at[pl.ds(s * rpt, rpt)])
    pltpu.sync_copy(zeros_hbm.at[pl.ds(s * rpt, rpt)],
                    acc_i.at[pl.ds(s * rpt, rpt)])
    pltpu.async_copy(ones_hbm, ones_v, sem).wait()
    pltpu.sync_copy(srcm.at[wid], src_v)
    pltpu.sync_copy(dstm.at[wid], dst_v)
    pltpu.sync_copy(srct.at[wid], st_v)
    pltpu.sync_copy(dstt.at[wid], dt_v)
    plsc.subcore_barrier()

    def body(i, carry):
        j = 2 * i
        d1 = pltpu.async_copy(ones_v, acc_o.at[src_v.at[j]], sem, add=True)
        d2 = pltpu.async_copy(ones_v, acc_i.at[dst_v.at[j]], sem2, add=True)
        d3 = pltpu.async_copy(ones_v, acc_o.at[src_v.at[j + 1]], sem3,
                              add=True)
        d4 = pltpu.async_copy(ones_v, acc_i.at[dst_v.at[j + 1]], sem4,
                              add=True)
        d1.wait()
        d2.wait()
        d3.wait()
        d4.wait()
        return carry

    lax.fori_loop(0, full // 2, body, 0, unroll=False)
    for j in range(full // 2 * 2, full):
        d1 = pltpu.async_copy(ones_v, acc_o.at[src_v.at[j]], sem, add=True)
        d2 = pltpu.async_copy(ones_v, acc_i.at[dst_v.at[j]], sem2, add=True)
        d1.wait()
        d2.wait()
    d1 = pltpu.async_copy(ones_v.at[pl.ds(0, st_v.shape[0])],
                          acc_o.at[st_v], sem, add=True)
    d2 = pltpu.async_copy(ones_v.at[pl.ds(0, dt_v.shape[0])],
                          acc_i.at[dt_v], sem2, add=True)
    d1.wait()
    d2.wait()
    plsc.subcore_barrier()

    pltpu.sync_copy(acc_o.at[pl.ds(s * rpt, rpt)],
                    out_hbm.at[c, 0, pl.ds(s * rpt, rpt), pl.ds(0, _DW)])
    pltpu.sync_copy(acc_o.at[pl.ds(s * rpt, rpt)],
                    out_hbm.at[c, 0, pl.ds(s * rpt, rpt), pl.ds(_DW, _DW)])
    pltpu.sync_copy(acc_i.at[pl.ds(s * rpt, rpt)],
                    out_hbm.at[c, 1, pl.ds(s * rpt, rpt), pl.ds(0, _DW)])
    pltpu.sync_copy(acc_i.at[pl.ds(s * rpt, rpt)],
                    out_hbm.at[c, 1, pl.ds(s * rpt, rpt), pl.ds(_DW, _DW)])


def _make_deg_kernel(np_, full, rem):
    mesh = plsc.VectorSubcoreMesh(core_axis_name="c", subcore_axis_name="s")
    return pl.kernel(
        _deg_body,
        out_type=jax.ShapeDtypeStruct((_NC, 2, np_, _H), jnp.float32),
        mesh=mesh,
        compiler_params=pltpu.CompilerParams(use_tc_tiling_on_sc=False),
        scratch_types=[
            pltpu.VMEM((_CHUNK, _DW), jnp.float32),
            pltpu.VMEM((full, _CHUNK), jnp.int32),
            pltpu.VMEM((full, _CHUNK), jnp.int32),
            pltpu.VMEM((rem,), jnp.int32),
            pltpu.VMEM((rem,), jnp.int32),
            pltpu.VMEM_SHARED((np_, _DW), jnp.float32),
            pltpu.VMEM_SHARED((np_, _DW), jnp.float32),
            pltpu.SemaphoreType.DMA,
            pltpu.SemaphoreType.DMA,
            pltpu.SemaphoreType.DMA,
            pltpu.SemaphoreType.DMA,
        ],
    )


def _agg_body(g_hbm, srcm, dstm, srct, dstt, zeros_hbm, out_hbm,
              src_v, dst_v, st_v, dt_v, msgs, msg_t, gsems, ssems, acc):
    c = lax.axis_index("c")
    s = lax.axis_index("s")
    wid = c * _NS + s
    np_ = acc.shape[0]
    rpt = np_ // _NS
    full = src_v.shape[0]
    pltpu.sync_copy(zeros_hbm.at[pl.ds(s * rpt, rpt)],
                    acc.at[pl.ds(s * rpt, rpt)])
    pltpu.async_copy(srcm.at[wid], src_v, gsems[0]).wait()
    pltpu.async_copy(dstm.at[wid], dst_v, gsems[1]).wait()
    pltpu.async_copy(srct.at[wid], st_v, gsems[2]).wait()
    pltpu.async_copy(dstt.at[wid], dt_v, gsems[3]).wait()
    plsc.subcore_barrier()

    for k in range(_NBUF):
        pltpu.async_copy(g_hbm.at[src_v.at[k]], msgs[k], gsems[k])

    main = full // _NBUF * _NBUF

    def body(i, carry):
        j = _NBUF * i
        descs = []
        for k in range(_NBUF):
            pltpu.make_async_copy(g_hbm.at[src_v.at[j + k]],
                                  msgs[k], gsems[k]).wait()
            descs.append(pltpu.async_copy(msgs[k], acc.at[dst_v.at[j + k]],
                                          ssems[k], add=True))
        for k in range(_NBUF):
            descs[k].wait()

            @pl.when(j + _NBUF + k < full)
            def _(k=k, j=j):
                jn = jnp.minimum(j + _NBUF + k, full - 1)
                pltpu.async_copy(g_hbm.at[src_v.at[jn]], msgs[k], gsems[k])

        return carry

    lax.fori_loop(0, full // _NBUF, body, 0, unroll=False)
    for j in range(main, full):
        k = j % _NBUF
        pltpu.make_async_copy(g_hbm.at[src_v.at[j]], msgs[k], gsems[k]).wait()
        pltpu.sync_copy(msgs[k], acc.at[dst_v.at[j]], add=True)
    pltpu.async_copy(g_hbm.at[st_v], msg_t, gsems[0]).wait()
    pltpu.sync_copy(msg_t, acc.at[dt_v], add=True)
    plsc.subcore_barrier()

    pltpu.sync_copy(acc.at[pl.ds(s * rpt, rpt)],
                    out_hbm.at[c, pl.ds(s * rpt, rpt)])


def _make_agg_kernel(np_, full, rem):
    mesh = plsc.VectorSubcoreMesh(core_axis_name="c", subcore_axis_name="s")
    return pl.kernel(
        _agg_body,
        out_type=jax.ShapeDtypeStruct((_NC, np_, _H), jnp.float32),
        mesh=mesh,
        compiler_params=pltpu.CompilerParams(use_tc_tiling_on_sc=False),
        scratch_types=[
            pltpu.VMEM((full, _CHUNK), jnp.int32),
            pltpu.VMEM((full, _CHUNK), jnp.int32),
            pltpu.VMEM((rem,), jnp.int32),
            pltpu.VMEM((rem,), jnp.int32),
            [pltpu.VMEM((_CHUNK, _H), jnp.float32)] * _NBUF,
            pltpu.VMEM((rem, _H), jnp.float32),
            [pltpu.SemaphoreType.DMA] * _NBUF,
            [pltpu.SemaphoreType.DMA] * _NBUF,
            pltpu.VMEM_SHARED((np_, _H), jnp.float32),
        ],
    )


def _norm(deg):
    return jnp.where(deg > 0.0, lax.rsqrt(jnp.maximum(deg, 1.0)), 0.0)


def _tc_head_body(x_ref, w1r_ref, mask_ref, degp_ref, g0_ref):
    t2 = jnp.dot(x_ref[...], w1r_ref[...], preferred_element_type=jnp.float32)
    n8 = t2.shape[0] // 8
    packed = jnp.sum(t2.reshape(n8, 8, 128) * mask_ref[...], axis=1)
    don_r = _norm(degp_ref[0, 0] + degp_ref[1, 0])
    m = g0_ref.shape[0]
    g0_ref[...] = jnp.concatenate(
        [packed, jnp.zeros((m - n8, 128), jnp.float32)]) * don_r


def _tc_mid_body(p_ref, degp_ref, wb_ref, b_ref, g_ref):
    don_r = _norm(degp_ref[0, 0] + degp_ref[1, 0])
    din_r = _norm(degp_ref[0, 1] + degp_ref[1, 1])
    hidden = jnp.maximum(din_r * (p_ref[0] + p_ref[1]) + b_ref[...], 0.0)
    g_ref[...] = jnp.dot(hidden, wb_ref[...],
                         preferred_element_type=jnp.float32) * don_r


def _tc_tail_body(p_ref, degp_ref, b_ref, out_ref):
    din_r = _norm(degp_ref[0, 1] + degp_ref[1, 1])
    out_ref[...] = din_r * (p_ref[0] + p_ref[1]) + b_ref[...]


@jax.jit
def kernel(x, edge_index, W1, b1, W2, b2, W3, b3):
    n, f = x.shape
    h = W1.shape[1]
    out_w = W3.shape[1]
    e = edge_index.shape[1]

    np_ = ((n + _NT * 8 - 1) // (_NT * 8)) * (_NT * 8)
    m = np_ // 8
    pe = e // _NT
    full = pe // _CHUNK
    rem = pe - full * _CHUNK
    e_main = _NT * full * _CHUNK

    srcm = edge_index[0, :e_main].reshape(_NT, full, _CHUNK)
    dstm = edge_index[1, :e_main].reshape(_NT, full, _CHUNK)
    srct = edge_index[0, e_main:].reshape(_NT, rem)
    dstt = edge_index[1, e_main:].reshape(_NT, rem)

    zeros16 = jnp.zeros((np_, _H), jnp.float32)
    zeros8 = jnp.zeros((np_, _DW), jnp.float32)
    ones_d = jnp.ones((_CHUNK, _DW), jnp.float32)
    eye8 = jnp.eye(8, dtype=jnp.float32)
    w1rep = jnp.tile(W1, (1, 8))
    maskc = jnp.kron(eye8, jnp.ones((1, h), jnp.float32))
    w2big = jnp.kron(eye8, W2)
    w3p = jnp.pad(W3, ((0, 0), (0, h - out_w)))
    w3big = jnp.kron(eye8, w3p)
    b1p = jnp.tile(b1, 8).reshape(1, 128)
    b2p = jnp.tile(b2, 8).reshape(1, 128)
    b3p = jnp.tile(jnp.pad(b3, (0, h - out_w)), 8).reshape(1, 128)

    degp = _make_deg_kernel(np_, full, rem)(srcm, dstm, srct, dstt,
                                            ones_d, zeros8)
    degp_r = degp.reshape(_NC, 2, m, 128)

    g0 = pl.pallas_call(
        _tc_head_body,
        out_shape=jax.ShapeDtypeStruct((m, 128), jnp.float32),
    )(x, w1rep, maskc, degp_r)

    agg = _make_agg_kernel(np_, full, rem)

    p1 = agg(g0.reshape(np_, _H), srcm, dstm, srct, dstt, zeros16)
    g1 = pl.pallas_call(
        _tc_mid_body,
        out_shape=jax.ShapeDtypeStruct((m, 128), jnp.float32),
    )(p1.reshape(_NC, m, 128), degp_r, w2big, b1p)

    p2 = agg(g1.reshape(np_, _H), srcm, dstm, srct, dstt, zeros16)
    g2 = pl.pallas_call(
        _tc_mid_body,
        out_shape=jax.ShapeDtypeStruct((m, 128), jnp.float32),
    )(p2.reshape(_NC, m, 128), degp_r, w3big, b2p)

    p3 = agg(g2.reshape(np_, _H), srcm, dstm, srct, dstt, zeros16)
    y_pack = pl.pallas_call(
        _tc_tail_body,
        out_shape=jax.ShapeDtypeStruct((m, 128), jnp.float32),
    )(p3.reshape(_NC, m, 128), degp_r, b3p)

    return y_pack.reshape(np_, _H)[:n, :out_w]


# speedup vs baseline: 1.0741x; 1.0741x over previous
import jax
import jax.numpy as jnp
from jax import lax
from jax.experimental import pallas as pl
from jax.experimental.pallas import tpu as pltpu
from jax.experimental.pallas import tpu_sc as plsc

_INFO = plsc.get_sparse_core_info()
_NC = _INFO.num_cores
_NS = _INFO.num_subcores
_NT = _NC * _NS
_CHUNK = 128
_NBUF = 8
_H = 16


def _deg_body(srcm, dstm, srct, dstt, ones_hbm, zeros_hbm, out_hbm,
              ones_v, src_v, dst_v, st_v, dt_v, acc_o, acc_i,
              sem, sem2, sem3, sem4):
    c = lax.axis_index("c")
    s = lax.axis_index("s")
    wid = c * _NS + s
    np_ = acc_o.shape[0]
    rpt = np_ // _NS
    full = src_v.shape[0]

    pltpu.sync_copy(zeros_hbm.at[pl.ds(s * rpt, rpt)],
                    acc_o.at[pl.ds(s * rpt, rpt)])
    pltpu.sync_copy(zeros_hbm.at[pl.ds(s * rpt, rpt)],
                    acc_i.at[pl.ds(s * rpt, rpt)])
    pltpu.async_copy(ones_hbm, ones_v, sem).wait()
    pltpu.sync_copy(srcm.at[wid], src_v)
    pltpu.sync_copy(dstm.at[wid], dst_v)
    pltpu.sync_copy(srct.at[wid], st_v)
    pltpu.sync_copy(dstt.at[wid], dt_v)
    plsc.subcore_barrier()

    def body(i, carry):
        j = 2 * i
        d1 = pltpu.async_copy(ones_v, acc_o.at[src_v.at[j]], sem, add=True)
        d2 = pltpu.async_copy(ones_v, acc_i.at[dst_v.at[j]], sem2, add=True)
        d3 = pltpu.async_copy(ones_v, acc_o.at[src_v.at[j + 1]], sem3,
                              add=True)
        d4 = pltpu.async_copy(ones_v, acc_i.at[dst_v.at[j + 1]], sem4,
                              add=True)
        d1.wait()
        d2.wait()
        d3.wait()
        d4.wait()
        return carry

    lax.fori_loop(0, full // 2, body, 0, unroll=False)
    for j in range(full // 2 * 2, full):
        d1 = pltpu.async_copy(ones_v, acc_o.at[src_v.at[j]], sem, add=True)
        d2 = pltpu.async_copy(ones_v, acc_i.at[dst_v.at[j]], sem2, add=True)
        d1.wait()
        d2.wait()
    d1 = pltpu.async_copy(ones_v.at[pl.ds(0, st_v.shape[0])],
                          acc_o.at[st_v], sem, add=True)
    d2 = pltpu.async_copy(ones_v.at[pl.ds(0, dt_v.shape[0])],
                          acc_i.at[dt_v], sem2, add=True)
    d1.wait()
    d2.wait()
    plsc.subcore_barrier()

    pltpu.sync_copy(acc_o.at[pl.ds(s * rpt, rpt)],
                    out_hbm.at[c, 0, pl.ds(s * rpt, rpt)])
    pltpu.sync_copy(acc_i.at[pl.ds(s * rpt, rpt)],
                    out_hbm.at[c, 1, pl.ds(s * rpt, rpt)])


def _make_deg_kernel(np_, full, rem):
    mesh = plsc.VectorSubcoreMesh(core_axis_name="c", subcore_axis_name="s")
    return pl.kernel(
        _deg_body,
        out_type=jax.ShapeDtypeStruct((_NC, 2, np_, _H), jnp.float32),
        mesh=mesh,
        compiler_params=pltpu.CompilerParams(use_tc_tiling_on_sc=False),
        scratch_types=[
            pltpu.VMEM((_CHUNK, _H), jnp.float32),
            pltpu.VMEM((full, _CHUNK), jnp.int32),
            pltpu.VMEM((full, _CHUNK), jnp.int32),
            pltpu.VMEM((rem,), jnp.int32),
            pltpu.VMEM((rem,), jnp.int32),
            pltpu.VMEM_SHARED((np_, _H), jnp.float32),
            pltpu.VMEM_SHARED((np_, _H), jnp.float32),
            pltpu.SemaphoreType.DMA,
            pltpu.SemaphoreType.DMA,
            pltpu.SemaphoreType.DMA,
            pltpu.SemaphoreType.DMA,
        ],
    )


def _agg_body(g_hbm, srcm, dstm, srct, dstt, zeros_hbm, out_hbm,
              src_v, dst_v, st_v, dt_v, msgs, msg_t, gsems, ssems, acc):
    c = lax.axis_index("c")
    s = lax.axis_index("s")
    wid = c * _NS + s
    np_ = acc.shape[0]
    rpt = np_ // _NS
    full = src_v.shape[0]

    pltpu.sync_copy(zeros_hbm.at[pl.ds(s * rpt, rpt)],
                    acc.at[pl.ds(s * rpt, rpt)])
    pltpu.async_copy(srcm.at[wid], src_v, gsems[0]).wait()
    pltpu.async_copy(dstm.at[wid], dst_v, gsems[1]).wait()
    pltpu.async_copy(srct.at[wid], st_v, gsems[2]).wait()
    pltpu.async_copy(dstt.at[wid], dt_v, gsems[3]).wait()
    plsc.subcore_barrier()

    for k in range(_NBUF):
        pltpu.async_copy(g_hbm.at[src_v.at[k]], msgs[k], gsems[k])

    main = full // _NBUF * _NBUF

    def body(i, carry):
        j = _NBUF * i
        descs = []
        for k in range(_NBUF):
            pltpu.make_async_copy(g_hbm.at[src_v.at[j + k]],
                                  msgs[k], gsems[k]).wait()
            descs.append(pltpu.async_copy(msgs[k], acc.at[dst_v.at[j + k]],
                                          ssems[k], add=True))
        for k in range(_NBUF):
            descs[k].wait()

            @pl.when(j + _NBUF + k < full)
            def _(k=k, j=j):
                jn = jnp.minimum(j + _NBUF + k, full - 1)
                pltpu.async_copy(g_hbm.at[src_v.at[jn]], msgs[k], gsems[k])

        return carry

    lax.fori_loop(0, full // _NBUF, body, 0, unroll=False)
    for j in range(main, full):
        k = j % _NBUF
        pltpu.make_async_copy(g_hbm.at[src_v.at[j]], msgs[k], gsems[k]).wait()
        pltpu.sync_copy(msgs[k], acc.at[dst_v.at[j]], add=True)
    pltpu.async_copy(g_hbm.at[st_v], msg_t, gsems[0]).wait()
    pltpu.sync_copy(msg_t, acc.at[dt_v], add=True)
    plsc.subcore_barrier()

    pltpu.sync_copy(acc.at[pl.ds(s * rpt, rpt)],
                    out_hbm.at[c, pl.ds(s * rpt, rpt)])


def _make_agg_kernel(np_, full, rem):
    mesh = plsc.VectorSubcoreMesh(core_axis_name="c", subcore_axis_name="s")
    return pl.kernel(
        _agg_body,
        out_type=jax.ShapeDtypeStruct((_NC, np_, _H), jnp.float32),
        mesh=mesh,
        compiler_params=pltpu.CompilerParams(use_tc_tiling_on_sc=False),
        scratch_types=[
            pltpu.VMEM((full, _CHUNK), jnp.int32),
            pltpu.VMEM((full, _CHUNK), jnp.int32),
            pltpu.VMEM((rem,), jnp.int32),
            pltpu.VMEM((rem,), jnp.int32),
            [pltpu.VMEM((_CHUNK, _H), jnp.float32)] * _NBUF,
            pltpu.VMEM((rem, _H), jnp.float32),
            [pltpu.SemaphoreType.DMA] * _NBUF,
            [pltpu.SemaphoreType.DMA] * _NBUF,
            pltpu.VMEM_SHARED((np_, _H), jnp.float32),
        ],
    )


def _norm(deg):
    return jnp.where(deg > 0.0, lax.rsqrt(jnp.maximum(deg, 1.0)), 0.0)


def _tc_head_body(x_ref, w1r_ref, mask_ref, degp_ref, g0_ref):
    t2 = jnp.dot(x_ref[...], w1r_ref[...], preferred_element_type=jnp.float32)
    n8 = t2.shape[0] // 8
    packed = jnp.sum(t2.reshape(n8, 8, 128) * mask_ref[...], axis=1)
    don_r = _norm(degp_ref[0, 0] + degp_ref[1, 0])
    m = g0_ref.shape[0]
    g0_ref[...] = jnp.concatenate(
        [packed, jnp.zeros((m - n8, 128), jnp.float32)]) * don_r


def _tc_mid_body(p_ref, degp_ref, wb_ref, b_ref, g_ref):
    don_r = _norm(degp_ref[0, 0] + degp_ref[1, 0])
    din_r = _norm(degp_ref[0, 1] + degp_ref[1, 1])
    hidden = jnp.maximum(din_r * (p_ref[0] + p_ref[1]) + b_ref[...], 0.0)
    g_ref[...] = jnp.dot(hidden, wb_ref[...],
                         preferred_element_type=jnp.float32) * don_r


def _tc_tail_body(p_ref, degp_ref, b_ref, out_ref):
    din_r = _norm(degp_ref[0, 1] + degp_ref[1, 1])
    out_ref[...] = din_r * (p_ref[0] + p_ref[1]) + b_ref[...]


@jax.jit
def kernel(x, edge_index, W1, b1, W2, b2, W3, b3):
    n, f = x.shape
    h = W1.shape[1]
    out_w = W3.shape[1]
    e = edge_index.shape[1]

    np_ = ((n + _NT * 8 - 1) // (_NT * 8)) * (_NT * 8)
    m = np_ // 8
    pe = e // _NT
    full = pe // _CHUNK
    rem = pe - full * _CHUNK
    e_main = _NT * full * _CHUNK

    srcm = edge_index[0, :e_main].reshape(_NT, full, _CHUNK)
    dstm = edge_index[1, :e_main].reshape(_NT, full, _CHUNK)
    srct = edge_index[0, e_main:].reshape(_NT, rem)
    dstt = edge_index[1, e_main:].reshape(_NT, rem)

    zeros16 = jnp.zeros((np_, _H), jnp.float32)
    ones_c = jnp.ones((_CHUNK, _H), jnp.float32)
    eye8 = jnp.eye(8, dtype=jnp.float32)
    w1rep = jnp.tile(W1, (1, 8))
    maskc = jnp.kron(eye8, jnp.ones((1, h), jnp.float32))
    w2big = jnp.kron(eye8, W2)
    w3p = jnp.pad(W3, ((0, 0), (0, h - out_w)))
    w3big = jnp.kron(eye8, w3p)
    b1p = jnp.tile(b1, 8).reshape(1, 128)
    b2p = jnp.tile(b2, 8).reshape(1, 128)
    b3p = jnp.tile(jnp.pad(b3, (0, h - out_w)), 8).reshape(1, 128)

    degp = _make_deg_kernel(np_, full, rem)(srcm, dstm, srct, dstt,
                                            ones_c, zeros16)
    degp_r = degp.reshape(_NC, 2, m, 128)

    g0 = pl.pallas_call(
        _tc_head_body,
        out_shape=jax.ShapeDtypeStruct((m, 128), jnp.float32),
    )(x, w1rep, maskc, degp_r)

    agg = _make_agg_kernel(np_, full, rem)

    p1 = agg(g0.reshape(np_, _H), srcm, dstm, srct, dstt, zeros16)
    g1 = pl.pallas_call(
        _tc_mid_body,
        out_shape=jax.ShapeDtypeStruct((m, 128), jnp.float32),
    )(p1.reshape(_NC, m, 128), degp_r, w2big, b1p)

    p2 = agg(g1.reshape(np_, _H), srcm, dstm, srct, dstt, zeros16)
    g2 = pl.pallas_call(
        _tc_mid_body,
        out_shape=jax.ShapeDtypeStruct((m, 128), jnp.float32),
    )(p2.reshape(_NC, m, 128), degp_r, w3big, b2p)

    p3 = agg(g2.reshape(np_, _H), srcm, dstm, srct, dstt, zeros16)
    y_pack = pl.pallas_call(
        _tc_tail_body,
        out_shape=jax.ShapeDtypeStruct((m, 128), jnp.float32),
    )(p3.reshape(_NC, m, 128), degp_r, b3p)

    return y_pack.reshape(np_, _H)[:n, :out_w]


# speedup vs baseline: 1.0889x; 1.0138x over previous
import jax
import jax.numpy as jnp
from jax import lax
from jax.experimental import pallas as pl
from jax.experimental.pallas import tpu as pltpu
from jax.experimental.pallas import tpu_sc as plsc

_INFO = plsc.get_sparse_core_info()
_NC = _INFO.num_cores
_NS = _INFO.num_subcores
_NT = _NC * _NS
_CHUNK = 128
_NBUF = 8
_H = 16


def _deg_body(srcm, dstm, srct, dstt, ones_hbm, zeros_hbm, out_hbm,
              ones_v, src_v, dst_v, st_v, dt_v, acc_o, acc_i,
              sem, sem2, sem3, sem4):
    c = lax.axis_index("c")
    s = lax.axis_index("s")
    wid = c * _NS + s
    np_ = acc_o.shape[0]
    rpt = np_ // _NS
    full = src_v.shape[0]

    pltpu.sync_copy(zeros_hbm.at[pl.ds(s * rpt, rpt)],
                    acc_o.at[pl.ds(s * rpt, rpt)])
    pltpu.sync_copy(zeros_hbm.at[pl.ds(s * rpt, rpt)],
                    acc_i.at[pl.ds(s * rpt, rpt)])
    pltpu.async_copy(ones_hbm, ones_v, sem).wait()
    pltpu.sync_copy(srcm.at[wid], src_v)
    pltpu.sync_copy(dstm.at[wid], dst_v)
    pltpu.sync_copy(srct.at[wid], st_v)
    pltpu.sync_copy(dstt.at[wid], dt_v)
    plsc.subcore_barrier()

    def body(i, carry):
        j = 2 * i
        d1 = pltpu.async_copy(ones_v, acc_o.at[src_v.at[j]], sem, add=True)
        d2 = pltpu.async_copy(ones_v, acc_i.at[dst_v.at[j]], sem2, add=True)
        d3 = pltpu.async_copy(ones_v, acc_o.at[src_v.at[j + 1]], sem3,
                              add=True)
        d4 = pltpu.async_copy(ones_v, acc_i.at[dst_v.at[j + 1]], sem4,
                              add=True)
        d1.wait()
        d2.wait()
        d3.wait()
        d4.wait()
        return carry

    lax.fori_loop(0, full // 2, body, 0, unroll=False)
    for j in range(full // 2 * 2, full):
        d1 = pltpu.async_copy(ones_v, acc_o.at[src_v.at[j]], sem, add=True)
        d2 = pltpu.async_copy(ones_v, acc_i.at[dst_v.at[j]], sem2, add=True)
        d1.wait()
        d2.wait()
    d1 = pltpu.async_copy(ones_v.at[pl.ds(0, st_v.shape[0])],
                          acc_o.at[st_v], sem, add=True)
    d2 = pltpu.async_copy(ones_v.at[pl.ds(0, dt_v.shape[0])],
                          acc_i.at[dt_v], sem2, add=True)
    d1.wait()
    d2.wait()
    plsc.subcore_barrier()

    pltpu.sync_copy(acc_o.at[pl.ds(s * rpt, rpt)],
                    out_hbm.at[c, 0, pl.ds(s * rpt, rpt)])
    pltpu.sync_copy(acc_i.at[pl.ds(s * rpt, rpt)],
                    out_hbm.at[c, 1, pl.ds(s * rpt, rpt)])


def _make_deg_kernel(np_, full, rem):
    mesh = plsc.VectorSubcoreMesh(core_axis_name="c", subcore_axis_name="s")
    return pl.kernel(
        _deg_body,
        out_type=jax.ShapeDtypeStruct((_NC, 2, np_, _H), jnp.float32),
        mesh=mesh,
        compiler_params=pltpu.CompilerParams(use_tc_tiling_on_sc=False),
        scratch_types=[
            pltpu.VMEM((_CHUNK, _H), jnp.float32),
            pltpu.VMEM((full, _CHUNK), jnp.int32),
            pltpu.VMEM((full, _CHUNK), jnp.int32),
            pltpu.VMEM((rem,), jnp.int32),
            pltpu.VMEM((rem,), jnp.int32),
            pltpu.VMEM_SHARED((np_, _H), jnp.float32),
            pltpu.VMEM_SHARED((np_, _H), jnp.float32),
            pltpu.SemaphoreType.DMA,
            pltpu.SemaphoreType.DMA,
            pltpu.SemaphoreType.DMA,
            pltpu.SemaphoreType.DMA,
        ],
    )


def _agg_body(g_hbm, srcm, dstm, srct, dstt, zeros_hbm, out_hbm,
              src_v, dst_v, st_v, dt_v, msgs, msg_t, gsems, ssems, acc):
    c = lax.axis_index("c")
    s = lax.axis_index("s")
    wid = c * _NS + s
    np_ = acc.shape[0]
    rpt = np_ // _NS
    full = src_v.shape[0]

    pltpu.sync_copy(zeros_hbm.at[pl.ds(s * rpt, rpt)],
                    acc.at[pl.ds(s * rpt, rpt)])
    pltpu.async_copy(srcm.at[wid], src_v, gsems[0]).wait()
    pltpu.async_copy(dstm.at[wid], dst_v, gsems[1]).wait()
    pltpu.async_copy(srct.at[wid], st_v, gsems[2]).wait()
    pltpu.async_copy(dstt.at[wid], dt_v, gsems[3]).wait()
    plsc.subcore_barrier()

    for k in range(_NBUF):
        pltpu.async_copy(g_hbm.at[src_v.at[k]], msgs[k], gsems[k])

    main = full // _NBUF * _NBUF

    def body(i, carry):
        j = _NBUF * i
        descs = []
        for k in range(_NBUF):
            pltpu.make_async_copy(g_hbm.at[src_v.at[j + k]],
                                  msgs[k], gsems[k]).wait()
            descs.append(pltpu.async_copy(msgs[k], acc.at[dst_v.at[j + k]],
                                          ssems[k], add=True))
        for k in range(_NBUF):
            descs[k].wait()

            @pl.when(j + _NBUF + k < full)
            def _(k=k, j=j):
                jn = jnp.minimum(j + _NBUF + k, full - 1)
                pltpu.async_copy(g_hbm.at[src_v.at[jn]], msgs[k], gsems[k])

        return carry

    lax.fori_loop(0, full // _NBUF, body, 0, unroll=False)
    for j in range(main, full):
        k = j % _NBUF
        pltpu.make_async_copy(g_hbm.at[src_v.at[j]], msgs[k], gsems[k]).wait()
        pltpu.sync_copy(msgs[k], acc.at[dst_v.at[j]], add=True)
    pltpu.async_copy(g_hbm.at[st_v], msg_t, gsems[0]).wait()
    pltpu.sync_copy(msg_t, acc.at[dt_v], add=True)
    plsc.subcore_barrier()

    pltpu.sync_copy(acc.at[pl.ds(s * rpt, rpt)],
                    out_hbm.at[c, pl.ds(s * rpt, rpt)])


def _make_agg_kernel(np_, full, rem):
    mesh = plsc.VectorSubcoreMesh(core_axis_name="c", subcore_axis_name="s")
    return pl.kernel(
        _agg_body,
        out_type=jax.ShapeDtypeStruct((_NC, np_, _H), jnp.float32),
        mesh=mesh,
        compiler_params=pltpu.CompilerParams(use_tc_tiling_on_sc=False),
        scratch_types=[
            pltpu.VMEM((full, _CHUNK), jnp.int32),
            pltpu.VMEM((full, _CHUNK), jnp.int32),
            pltpu.VMEM((rem,), jnp.int32),
            pltpu.VMEM((rem,), jnp.int32),
            [pltpu.VMEM((_CHUNK, _H), jnp.float32)] * _NBUF,
            pltpu.VMEM((rem, _H), jnp.float32),
            [pltpu.SemaphoreType.DMA] * _NBUF,
            [pltpu.SemaphoreType.DMA] * _NBUF,
            pltpu.VMEM_SHARED((np_, _H), jnp.float32),
        ],
    )


def _norm(deg):
    return jnp.where(deg > 0.0, lax.rsqrt(jnp.maximum(deg, 1.0)), 0.0)


def _tc_headmm_body(x_ref, w1r_ref, mask_ref, t_ref):
    t2 = jnp.dot(x_ref[...], w1r_ref[...], preferred_element_type=jnp.float32)
    n8 = t2.shape[0] // 8
    packed = jnp.sum(t2.reshape(n8, 8, 128) * mask_ref[...], axis=1)
    m = t_ref.shape[0]
    t_ref[...] = jnp.concatenate(
        [packed, jnp.zeros((m - n8, 128), jnp.float32)])


def _tc_head_body(t_ref, degp_ref, g0_ref):
    don_r = _norm(degp_ref[0, 0] + degp_ref[1, 0])
    g0_ref[...] = t_ref[...] * don_r


def _tc_mid_body(p_ref, degp_ref, wb_ref, b_ref, g_ref):
    don_r = _norm(degp_ref[0, 0] + degp_ref[1, 0])
    din_r = _norm(degp_ref[0, 1] + degp_ref[1, 1])
    hidden = jnp.maximum(din_r * (p_ref[0] + p_ref[1]) + b_ref[...], 0.0)
    g_ref[...] = jnp.dot(hidden, wb_ref[...],
                         preferred_element_type=jnp.float32) * don_r


def _tc_tail_body(p_ref, degp_ref, b_ref, out_ref):
    din_r = _norm(degp_ref[0, 1] + degp_ref[1, 1])
    out_ref[...] = din_r * (p_ref[0] + p_ref[1]) + b_ref[...]


@jax.jit
def kernel(x, edge_index, W1, b1, W2, b2, W3, b3):
    n, f = x.shape
    h = W1.shape[1]
    out_w = W3.shape[1]
    e = edge_index.shape[1]

    np_ = ((n + _NT * 8 - 1) // (_NT * 8)) * (_NT * 8)
    m = np_ // 8
    pe = e // _NT
    full = pe // _CHUNK
    rem = pe - full * _CHUNK
    e_main = _NT * full * _CHUNK

    srcm = edge_index[0, :e_main].reshape(_NT, full, _CHUNK)
    dstm = edge_index[1, :e_main].reshape(_NT, full, _CHUNK)
    srct = edge_index[0, e_main:].reshape(_NT, rem)
    dstt = edge_index[1, e_main:].reshape(_NT, rem)

    zeros16 = jnp.zeros((np_, _H), jnp.float32)
    ones_c = jnp.ones((_CHUNK, _H), jnp.float32)
    eye8 = jnp.eye(8, dtype=jnp.float32)
    w1rep = jnp.tile(W1, (1, 8))
    maskc = jnp.kron(eye8, jnp.ones((1, h), jnp.float32))
    w2big = jnp.kron(eye8, W2)
    w3p = jnp.pad(W3, ((0, 0), (0, h - out_w)))
    w3big = jnp.kron(eye8, w3p)
    b1p = jnp.tile(b1, 8).reshape(1, 128)
    b2p = jnp.tile(b2, 8).reshape(1, 128)
    b3p = jnp.tile(jnp.pad(b3, (0, h - out_w)), 8).reshape(1, 128)

    degp = _make_deg_kernel(np_, full, rem)(srcm, dstm, srct, dstt,
                                            ones_c, zeros16)
    degp_r = degp.reshape(_NC, 2, m, 128)

    t_pack = pl.pallas_call(
        _tc_headmm_body,
        out_shape=jax.ShapeDtypeStruct((m, 128), jnp.float32),
    )(x, w1rep, maskc)
    g0 = pl.pallas_call(
        _tc_head_body,
        out_shape=jax.ShapeDtypeStruct((m, 128), jnp.float32),
    )(t_pack, degp_r)

    agg = _make_agg_kernel(np_, full, rem)

    p1 = agg(g0.reshape(np_, _H), srcm, dstm, srct, dstt, zeros16)
    g1 = pl.pallas_call(
        _tc_mid_body,
        out_shape=jax.ShapeDtypeStruct((m, 128), jnp.float32),
    )(p1.reshape(_NC, m, 128), degp_r, w2big, b1p)

    p2 = agg(g1.reshape(np_, _H), srcm, dstm, srct, dstt, zeros16)
    g2 = pl.pallas_call(
        _tc_mid_body,
        out_shape=jax.ShapeDtypeStruct((m, 128), jnp.float32),
    )(p2.reshape(_NC, m, 128), degp_r, w3big, b2p)

    p3 = agg(g2.reshape(np_, _H), srcm, dstm, srct, dstt, zeros16)
    y_pack = pl.pallas_call(
        _tc_tail_body,
        out_shape=jax.ShapeDtypeStruct((m, 128), jnp.float32),
    )(p3.reshape(_NC, m, 128), degp_r, b3p)

    return y_pack.reshape(np_, _H)[:n, :out_w]
